# baseline (speedup 1.0000x reference)
import functools

import jax
import jax.numpy as jnp
from jax import lax
from jax.experimental import pallas as pl
from jax.experimental.pallas import tpu as pltpu
from jax.experimental.pallas import tpu_sc as plsc


def kernel(x, pe_weight):
    b = x.shape[0]
    n, d = pe_weight.shape
    mesh = plsc.VectorSubcoreMesh(core_axis_name="c", subcore_axis_name="s")
    nw = mesh.num_cores * mesh.num_subcores
    rows = n // nw

    @functools.partial(
        pl.kernel,
        out_type=jax.ShapeDtypeStruct((b, n, d), pe_weight.dtype),
        mesh=mesh,
        scratch_types=[pltpu.SemaphoreType.DMA],
    )
    def sc_copy(pe_hbm, out_hbm, sem):
        wid = lax.axis_index("s") * mesh.num_cores + lax.axis_index("c")
        sl = pl.ds(wid * rows, rows)
        for i in range(b):
            pltpu.make_async_copy(pe_hbm.at[sl], out_hbm.at[i, sl], sem).start()
        for i in range(b):
            pltpu.make_async_copy(pe_hbm.at[sl], out_hbm.at[i, sl], sem).wait()

    return sc_copy(pe_weight)


# speedup vs baseline: 78.2821x; 78.2821x over previous
import jax
import jax.numpy as jnp
from jax.experimental import pallas as pl
from jax.experimental.pallas import tpu as pltpu

_N_CHUNKS = 16


def _make_body(b, n, d, n_chunks):
    rows = n // n_chunks

    def body(pe_hbm, out_hbm, vmem, in_sems, out_sems):
        for c in range(n_chunks):
            sl = pl.ds(c * rows, rows)
            pltpu.make_async_copy(pe_hbm.at[sl], vmem.at[sl], in_sems.at[c]).start()
        for c in range(n_chunks):
            sl = pl.ds(c * rows, rows)
            pltpu.make_async_copy(pe_hbm.at[sl], vmem.at[sl], in_sems.at[c]).wait()
            for i in range(b):
                pltpu.make_async_copy(
                    vmem.at[sl], out_hbm.at[i, sl], out_sems.at[c, i]
                ).start()
        for c in range(n_chunks):
            sl = pl.ds(c * rows, rows)
            for i in range(b):
                pltpu.make_async_copy(
                    vmem.at[sl], out_hbm.at[i, sl], out_sems.at[c, i]
                ).wait()

    return body


def kernel(x, pe_weight):
    b = x.shape[0]
    n, d = pe_weight.shape
    n_chunks = _N_CHUNKS if n % _N_CHUNKS == 0 else 1
    return pl.pallas_call(
        _make_body(b, n, d, n_chunks),
        in_specs=[pl.BlockSpec(memory_space=pl.ANY)],
        out_specs=pl.BlockSpec(memory_space=pl.ANY),
        out_shape=jax.ShapeDtypeStruct((b, n, d), pe_weight.dtype),
        scratch_shapes=[
            pltpu.VMEM((n, d), pe_weight.dtype),
            pltpu.SemaphoreType.DMA((n_chunks,)),
            pltpu.SemaphoreType.DMA((n_chunks, b)),
        ],
    )(pe_weight)


# speedup vs baseline: 81.7493x; 1.0443x over previous
import jax
import jax.numpy as jnp
from jax.experimental import pallas as pl
from jax.experimental.pallas import tpu as pltpu

_RAMP = (256, 256, 512, 1024, 2048)


def _chunks(n, ramp):
    if sum(ramp) == n:
        sizes = ramp
    else:
        sizes = (n // 4,) * 4
    offs, o = [], 0
    for s in sizes:
        offs.append((o, s))
        o += s
    return offs


def _make_body(b, n, d, chunks):
    n_chunks = len(chunks)

    def body(pe_hbm, out_hbm, vmem, in_sems, out_sems):
        for c, (o, s) in enumerate(chunks):
            sl = pl.ds(o, s)
            pltpu.make_async_copy(pe_hbm.at[sl], vmem.at[sl], in_sems.at[c]).start()
        for c, (o, s) in enumerate(chunks):
            sl = pl.ds(o, s)
            pltpu.make_async_copy(pe_hbm.at[sl], vmem.at[sl], in_sems.at[c]).wait()
            for i in range(b):
                pltpu.make_async_copy(
                    vmem.at[sl], out_hbm.at[i, sl], out_sems.at[c, i]
                ).start()
        for c, (o, s) in enumerate(chunks):
            sl = pl.ds(o, s)
            for i in range(b):
                pltpu.make_async_copy(
                    vmem.at[sl], out_hbm.at[i, sl], out_sems.at[c, i]
                ).wait()

    return body


def kernel(x, pe_weight):
    b = x.shape[0]
    n, d = pe_weight.shape
    chunks = _chunks(n, _RAMP)
    n_chunks = len(chunks)
    return pl.pallas_call(
        _make_body(b, n, d, chunks),
        in_specs=[pl.BlockSpec(memory_space=pl.ANY)],
        out_specs=pl.BlockSpec(memory_space=pl.ANY),
        out_shape=jax.ShapeDtypeStruct((b, n, d), pe_weight.dtype),
        scratch_shapes=[
            pltpu.VMEM((n, d), pe_weight.dtype),
            pltpu.SemaphoreType.DMA((n_chunks,)),
            pltpu.SemaphoreType.DMA((n_chunks, b)),
        ],
    )(pe_weight)
